# baseline (speedup 1.0000x reference)
import jax
import jax.numpy as jnp
from jax.experimental import pallas as pl
from jax.experimental.pallas import tpu as pltpu

TOPK = 8
N_EXPERTS = 64
HIDDEN = 2048
BLOCK_ROWS = 2048


def _gate_kernel(x1_ref, x2_ref, w1_ref, w2_ref, idx_ref, wgt_ref):
    logits = jax.lax.dot_general(
        x1_ref[...], w1_ref[...], (((1,), (0,)), ((), ())),
        preferred_element_type=jnp.float32,
    ) + jax.lax.dot_general(
        x2_ref[...], w2_ref[...], (((1,), (0,)), ((), ())),
        preferred_element_type=jnp.float32,
    )

    iota = jax.lax.broadcasted_iota(jnp.int32, logits.shape, 1)
    bits = jax.lax.bitcast_convert_type(logits, jnp.int32)
    code = jnp.where(bits < 0, iota, (N_EXPERTS - 1) - iota)
    packed = jax.lax.bitcast_convert_type(
        (bits & jnp.int32(~(N_EXPERTS - 1))) | code, jnp.float32
    )

    vals = []
    work = packed
    for k in range(TOPK):
        v = jnp.max(work, axis=-1, keepdims=True)
        vals.append(v)
        if k + 1 < TOPK:
            work = jnp.where(work == v, -jnp.inf, work)

    pk = jnp.concatenate(vals, axis=-1)
    pb = jax.lax.bitcast_convert_type(pk, jnp.int32)
    low = pb & jnp.int32(N_EXPERTS - 1)
    idx_ref[...] = jnp.where(pb < 0, low, (N_EXPERTS - 1) - low)
    topv = jax.lax.bitcast_convert_type(
        pb & jnp.int32(~(N_EXPERTS - 1)), jnp.float32
    )
    e = jnp.exp(topv - topv[:, :1])
    wgt_ref[...] = e / jnp.sum(e, axis=-1, keepdims=True)


def _gate(x, weight_t):
    n = x.shape[0]
    h2 = HIDDEN // 2
    grid = (n // BLOCK_ROWS,)
    idx, wgt = pl.pallas_call(
        _gate_kernel,
        grid=grid,
        in_specs=[
            pl.BlockSpec((BLOCK_ROWS, h2), lambda i: (i, 0)),
            pl.BlockSpec((BLOCK_ROWS, h2), lambda i: (i, 1)),
            pl.BlockSpec((h2, N_EXPERTS), lambda i: (0, 0)),
            pl.BlockSpec((h2, N_EXPERTS), lambda i: (1, 0)),
        ],
        out_specs=[
            pl.BlockSpec((BLOCK_ROWS, TOPK), lambda i: (i, 0)),
            pl.BlockSpec((BLOCK_ROWS, TOPK), lambda i: (i, 0)),
        ],
        out_shape=[
            jax.ShapeDtypeStruct((n, TOPK), jnp.int32),
            jax.ShapeDtypeStruct((n, TOPK), jnp.float32),
        ],
        compiler_params=pltpu.CompilerParams(
            dimension_semantics=("parallel",),
        ),
    )(x, x, weight_t, weight_t)
    return idx, wgt


def kernel(hidden_states, weight):
    b, s, h = hidden_states.shape
    x = hidden_states.reshape(-1, h)
    topk_idx, topk_weight = _gate(x, weight.T)
    aux_loss = jnp.array(0.0, dtype=jnp.float32)
    return (topk_idx, topk_weight, aux_loss)


# speedup vs baseline: 1.2348x; 1.2348x over previous
import jax
import jax.numpy as jnp
from jax.experimental import pallas as pl
from jax.experimental.pallas import tpu as pltpu

TOPK = 8
N_EXPERTS = 64
HIDDEN = 2048
BLOCK_ROWS = 2048
SUB_BLOCKS = 4


def _select8(logits):
    iota = jax.lax.broadcasted_iota(jnp.int32, logits.shape, 1)
    bits = jax.lax.bitcast_convert_type(logits, jnp.int32)
    code = jnp.where(bits < 0, iota, (N_EXPERTS - 1) - iota)
    packed = jax.lax.bitcast_convert_type(
        (bits & jnp.int32(~(N_EXPERTS - 1))) | code, jnp.float32
    )
    vals = []
    work = packed
    for k in range(TOPK):
        v = jnp.max(work, axis=-1, keepdims=True)
        vals.append(v)
        if k + 1 < TOPK:
            work = jnp.where(work == v, -jnp.inf, work)
    return jnp.concatenate(vals, axis=-1)


def _gate_kernel(x1_ref, x2_ref, w1_ref, w2_ref, idx_ref, wgt_ref):
    rows = BLOCK_ROWS // SUB_BLOCKS
    pks = []
    for c in range(SUB_BLOCKS):
        sl = slice(c * rows, (c + 1) * rows)
        logits = jax.lax.dot_general(
            x1_ref[sl, :], w1_ref[...], (((1,), (0,)), ((), ())),
            preferred_element_type=jnp.float32,
        ) + jax.lax.dot_general(
            x2_ref[sl, :], w2_ref[...], (((1,), (0,)), ((), ())),
            preferred_element_type=jnp.float32,
        )
        pks.append(_select8(logits))

    pk = jnp.concatenate(pks, axis=0)
    pb = jax.lax.bitcast_convert_type(pk, jnp.int32)
    low = pb & jnp.int32(N_EXPERTS - 1)
    idx_ref[...] = jnp.where(pb < 0, low, (N_EXPERTS - 1) - low)
    topv = jax.lax.bitcast_convert_type(
        pb & jnp.int32(~(N_EXPERTS - 1)), jnp.float32
    )
    e = jnp.exp(topv - topv[:, :1])
    wgt_ref[...] = e / jnp.sum(e, axis=-1, keepdims=True)


def _gate(x, weight_t):
    n = x.shape[0]
    h2 = HIDDEN // 2
    grid = (n // BLOCK_ROWS,)
    idx, wgt = pl.pallas_call(
        _gate_kernel,
        grid=grid,
        in_specs=[
            pl.BlockSpec((BLOCK_ROWS, h2), lambda i: (i, 0)),
            pl.BlockSpec((BLOCK_ROWS, h2), lambda i: (i, 1)),
            pl.BlockSpec((h2, N_EXPERTS), lambda i: (0, 0)),
            pl.BlockSpec((h2, N_EXPERTS), lambda i: (1, 0)),
        ],
        out_specs=[
            pl.BlockSpec((BLOCK_ROWS, TOPK), lambda i: (i, 0)),
            pl.BlockSpec((BLOCK_ROWS, TOPK), lambda i: (i, 0)),
        ],
        out_shape=[
            jax.ShapeDtypeStruct((n, TOPK), jnp.int32),
            jax.ShapeDtypeStruct((n, TOPK), jnp.float32),
        ],
        compiler_params=pltpu.CompilerParams(
            dimension_semantics=("parallel",),
        ),
    )(x, x, weight_t, weight_t)
    return idx, wgt


def kernel(hidden_states, weight):
    b, s, h = hidden_states.shape
    x = hidden_states.reshape(-1, h)
    topk_idx, topk_weight = _gate(x, weight.T)
    aux_loss = jnp.array(0.0, dtype=jnp.float32)
    return (topk_idx, topk_weight, aux_loss)


# speedup vs baseline: 1.2523x; 1.0142x over previous
import jax
import jax.numpy as jnp
from jax.experimental import pallas as pl
from jax.experimental.pallas import tpu as pltpu

TOPK = 8
N_EXPERTS = 64
HIDDEN = 2048
BLOCK_ROWS = 2048
SUB_BLOCKS = 8


def _select8(logits):
    iota = jax.lax.broadcasted_iota(jnp.int32, logits.shape, 1)
    bits = jax.lax.bitcast_convert_type(logits, jnp.int32)
    code = jnp.where(bits < 0, iota, (N_EXPERTS - 1) - iota)
    packed = jax.lax.bitcast_convert_type(
        (bits & jnp.int32(~(N_EXPERTS - 1))) | code, jnp.float32
    )
    vals = []
    work = packed
    for k in range(TOPK):
        v = jnp.max(work, axis=-1, keepdims=True)
        vals.append(v)
        if k + 1 < TOPK:
            work = jnp.where(work == v, -jnp.inf, work)
    return jnp.concatenate(vals, axis=-1)


def _gate_kernel(x1_ref, x2_ref, w1_ref, w2_ref, idx_ref, wgt_ref):
    rows = BLOCK_ROWS // SUB_BLOCKS
    pks = []
    for c in range(SUB_BLOCKS):
        sl = slice(c * rows, (c + 1) * rows)
        logits = jax.lax.dot_general(
            x1_ref[sl, :], w1_ref[...], (((1,), (0,)), ((), ())),
            preferred_element_type=jnp.float32,
        ) + jax.lax.dot_general(
            x2_ref[sl, :], w2_ref[...], (((1,), (0,)), ((), ())),
            preferred_element_type=jnp.float32,
        )
        pks.append(_select8(logits))

    pk = jnp.concatenate(pks, axis=0)
    pb = jax.lax.bitcast_convert_type(pk, jnp.int32)
    low = pb & jnp.int32(N_EXPERTS - 1)
    idx_ref[...] = jnp.where(pb < 0, low, (N_EXPERTS - 1) - low)
    topv = jax.lax.bitcast_convert_type(
        pb & jnp.int32(~(N_EXPERTS - 1)), jnp.float32
    )
    e = jnp.exp(topv - topv[:, :1])
    wgt_ref[...] = e / jnp.sum(e, axis=-1, keepdims=True)


def _gate(x, weight_t):
    n = x.shape[0]
    h2 = HIDDEN // 2
    grid = (n // BLOCK_ROWS,)
    idx, wgt = pl.pallas_call(
        _gate_kernel,
        grid=grid,
        in_specs=[
            pl.BlockSpec((BLOCK_ROWS, h2), lambda i: (i, 0)),
            pl.BlockSpec((BLOCK_ROWS, h2), lambda i: (i, 1)),
            pl.BlockSpec((h2, N_EXPERTS), lambda i: (0, 0)),
            pl.BlockSpec((h2, N_EXPERTS), lambda i: (1, 0)),
        ],
        out_specs=[
            pl.BlockSpec((BLOCK_ROWS, TOPK), lambda i: (i, 0)),
            pl.BlockSpec((BLOCK_ROWS, TOPK), lambda i: (i, 0)),
        ],
        out_shape=[
            jax.ShapeDtypeStruct((n, TOPK), jnp.int32),
            jax.ShapeDtypeStruct((n, TOPK), jnp.float32),
        ],
        compiler_params=pltpu.CompilerParams(
            dimension_semantics=("parallel",),
        ),
    )(x, x, weight_t, weight_t)
    return idx, wgt


def kernel(hidden_states, weight):
    b, s, h = hidden_states.shape
    x = hidden_states.reshape(-1, h)
    topk_idx, topk_weight = _gate(x, weight.T)
    aux_loss = jnp.array(0.0, dtype=jnp.float32)
    return (topk_idx, topk_weight, aux_loss)
